# Initial kernel scaffold; baseline (speedup 1.0000x reference)
import jax
import jax.numpy as jnp
from jax.experimental import pallas as pl


def kernel(x, spectral_prompt, W_spec, b_spec, conv_w, w_gate, fc1_w, fc1_b, fc2_w, fc2_b):
    raise NotImplementedError("write your pallas kernel here")


# speedup vs baseline: 7.6793x; 7.6793x over previous
import jax
import jax.numpy as jnp
from jax.experimental import pallas as pl
from jax.experimental.pallas import tpu as pltpu

_F32 = jnp.float32


def _emb_kernel(x_ref, emb_ref, *, inv_hw):
    t = pl.program_id(1)

    @pl.when(t == 0)
    def _():
        emb_ref[...] = jnp.zeros_like(emb_ref)

    emb_ref[0] += jnp.sum(x_ref[0], axis=1)[None, :] * inv_hw


def _moe_kernel(emb_ref, wspec_ref, bspec_ref, prompt_ref, convw_ref,
                wgate_ref, fc1w_ref, fc1bt_ref, fc2w_ref, fc2bt_ref, x_ref,
                out_ref, loss_ref, stats_acc, *, n_exp):
    b = pl.program_id(0)
    t = pl.program_id(1)
    nb = pl.num_programs(0)
    nt = pl.num_programs(1)

    @pl.when((b == 0) & (t == 0))
    def _():
        stats_acc[...] = jnp.zeros_like(stats_acc)

    xb = x_ref[0]

    emb = emb_ref[0]
    pl_log = jnp.dot(emb, wspec_ref[...].T,
                     preferred_element_type=_F32) + bspec_ref[...]
    pl_log = pl_log - jnp.max(pl_log, axis=1, keepdims=True)
    pe = jnp.exp(pl_log)
    pw = pe / jnp.sum(pe, axis=1, keepdims=True)
    spb = jnp.dot(pw, prompt_ref[...], preferred_element_type=_F32)

    m = convw_ref[...] * spb
    ot = jnp.dot(m, xb, preferred_element_type=_F32)

    le = jax.lax.dot_general(wgate_ref[...], xb, (((0,), (0,)), ((), ())),
                             preferred_element_type=_F32)
    eidx = jax.lax.broadcasted_iota(jnp.int32, le.shape, 0)
    l1 = jnp.max(le, axis=0, keepdims=True)
    i1 = jnp.min(jnp.where(le == l1, eidx, n_exp), axis=0, keepdims=True)
    masked = jnp.where(eidx == i1, -jnp.inf, le)
    l2 = jnp.max(masked, axis=0, keepdims=True)
    i2 = jnp.min(jnp.where(masked == l2, eidx, n_exp), axis=0, keepdims=True)
    ed = jnp.exp(l2 - l1)
    g1 = 1.0 / (1.0 + ed)
    g2 = ed / (1.0 + ed)
    gates = (jnp.where(eidx == i1, g1, 0.0)
             + jnp.where(eidx == i2, g2, 0.0))

    stats_acc[0:n_exp, 0:1] += jnp.sum(gates, axis=1, keepdims=True)
    stats_acc[n_exp:2 * n_exp, 0:1] += jnp.sum(
        (gates > 0.0).astype(_F32), axis=1, keepdims=True)

    y = xb
    for e in range(n_exp):
        h = jnp.dot(fc1w_ref[e], ot, preferred_element_type=_F32)
        h = h + fc1bt_ref[:, e:e + 1]
        h = 0.5 * h * (1.0 + jax.lax.erf(h * 0.7071067811865476))
        ye = jnp.dot(fc2w_ref[e], h, preferred_element_type=_F32)
        ye = ye + fc2bt_ref[:, e:e + 1]
        y = y + gates[e:e + 1, :] * ye
    out_ref[0] = y

    @pl.when((b == nb - 1) & (t == nt - 1))
    def _():
        def cv_sq(v):
            mean = jnp.sum(v) / n_exp
            var = jnp.sum((v - mean) ** 2) / (n_exp - 1)
            return var / (mean * mean + 1e-10)

        imp = stats_acc[0:n_exp, 0:1]
        load = stats_acc[n_exp:2 * n_exp, 0:1]
        loss = (cv_sq(imp) + cv_sq(load)) * 1e-2
        loss_ref[...] = jnp.full((1, 1), loss, dtype=_F32)


def _pick_tile(hw, target):
    best = hw
    for d in range(128, target + 1, 128):
        if hw % d == 0:
            best = d
    return best


def kernel(x, spectral_prompt, W_spec, b_spec, conv_w, w_gate,
           fc1_w, fc1_b, fc2_w, fc2_b):
    B, C, H, W = x.shape
    HW = H * W
    P = spectral_prompt.shape[0]
    E = w_gate.shape[1]
    hid = fc1_w.shape[1]
    hid_p = (hid + 127) // 128 * 128

    xr = x.reshape(B, C, HW)

    T1 = _pick_tile(HW, 7168)
    nt1 = HW // T1
    emb = pl.pallas_call(
        lambda x_ref, emb_ref: _emb_kernel(x_ref, emb_ref, inv_hw=1.0 / HW),
        grid=(B, nt1),
        in_specs=[pl.BlockSpec((1, C, T1), lambda b, t: (b, 0, t))],
        out_specs=pl.BlockSpec((1, 1, C), lambda b, t: (b, 0, 0)),
        out_shape=jax.ShapeDtypeStruct((B, 1, C), _F32),
        compiler_params=pltpu.CompilerParams(
            dimension_semantics=("arbitrary", "arbitrary")),
    )(xr)

    fc1_wp = jnp.pad(fc1_w, ((0, 0), (0, hid_p - hid), (0, 0)))
    fc2_wp = jnp.pad(fc2_w, ((0, 0), (0, 0), (0, hid_p - hid)))
    fc1_bt = jnp.pad(fc1_b, ((0, 0), (0, hid_p - hid))).T
    fc2_bt = fc2_b.T
    bspec2 = b_spec.reshape(1, P)

    T2 = _pick_tile(HW, 1792)
    nt2 = HW // T2

    out, loss = pl.pallas_call(
        lambda *refs: _moe_kernel(*refs, n_exp=E),
        grid=(B, nt2),
        in_specs=[
            pl.BlockSpec((1, 1, C), lambda b, t: (b, 0, 0)),
            pl.BlockSpec((P, C), lambda b, t: (0, 0)),
            pl.BlockSpec((1, P), lambda b, t: (0, 0)),
            pl.BlockSpec((P, C), lambda b, t: (0, 0)),
            pl.BlockSpec((C, C), lambda b, t: (0, 0)),
            pl.BlockSpec((C, E), lambda b, t: (0, 0)),
            pl.BlockSpec((E, hid_p, C), lambda b, t: (0, 0, 0)),
            pl.BlockSpec((hid_p, E), lambda b, t: (0, 0)),
            pl.BlockSpec((E, C, hid_p), lambda b, t: (0, 0, 0)),
            pl.BlockSpec((C, E), lambda b, t: (0, 0)),
            pl.BlockSpec((1, C, T2), lambda b, t: (b, 0, t)),
        ],
        out_specs=[
            pl.BlockSpec((1, C, T2), lambda b, t: (b, 0, t)),
            pl.BlockSpec((1, 1), lambda b, t: (0, 0)),
        ],
        out_shape=[
            jax.ShapeDtypeStruct((B, C, HW), _F32),
            jax.ShapeDtypeStruct((1, 1), _F32),
        ],
        scratch_shapes=[pltpu.VMEM((2 * E, 128), _F32)],
        compiler_params=pltpu.CompilerParams(
            dimension_semantics=("arbitrary", "arbitrary")),
    )(emb, W_spec, bspec2, spectral_prompt, conv_w, w_gate,
      fc1_wp, fc1_bt, fc2_wp, fc2_bt, xr)

    return out.reshape(B, C, H, W), loss[0, 0]
